# baseline (speedup 1.0000x reference)
import functools

import jax
import jax.numpy as jnp
import numpy as np
from jax import lax
from jax.experimental import pallas as pl
from jax.experimental.pallas import tpu as pltpu
from jax.experimental.pallas import tpu_sc as plsc

_B, _C, _H, _W = 4, 96, 384, 384
_N = _B * _H * _W
_SAMPLES_PER_IM = 5000
_EXPECTED = _SAMPLES_PER_IM * _B
_LAMBDS = (1.0 / 6.0, 5.0 / 6.0)
_ROW_BLOCK = 48

_NS, _LANES = 16, 16
_TILE_ELEMS = _N // _NS
_HALF_ELEMS = _TILE_ELEMS // 2
_STEPS = _HALF_ELEMS // _LANES
_BSHIFT = 12
_NBUCK = _N >> _BSHIFT
_BPAD = 160
_SUB = 1 << _BSHIFT


@functools.lru_cache(maxsize=1)
def _rank_constants():
    with jax.ensure_compile_time_eval():
        skey = jax.random.key(42)
        ka, kb = jax.random.split(skey)
        ranks = []
        for k in (ka, kb):
            g = jax.random.gumbel(k, (_N,), dtype=jnp.float32)
            perm = jnp.argsort(-g, stable=True)
            rank = jnp.zeros((_N,), jnp.int32).at[perm].set(
                jnp.arange(_N, dtype=jnp.int32))
            ranks.append(np.asarray(rank))
    return tuple(ranks)


def _lane_extract(vec, lane_idx):
    lanes = lax.iota(jnp.int32, _LANES)
    return jnp.sum(jnp.where(lanes == lane_idx, vec, 0))


def _sel_body(gt_hbm, ra_hbm, rb_hbm, out_hbm,
              gt_v, r_v, hist2d, hist1d, all_hist, bitmap, bm_all, ctl_v,
              sh_hist, sh_bm, sh_ctl):
    cid = lax.axis_index("c")
    sid = lax.axis_index("s")
    is_a = cid == 0
    lanes = lax.iota(jnp.int32, _LANES)
    ones = jnp.full((_LANES,), 1, jnp.int32)

    def load_half(h):
        base = sid * _TILE_ELEMS + h * _HALF_ELEMS
        pltpu.sync_copy(gt_hbm.at[pl.ds(base, _HALF_ELEMS)], gt_v)

        @pl.when(is_a)
        def _():
            pltpu.sync_copy(ra_hbm.at[pl.ds(base, _HALF_ELEMS)], r_v)

        @pl.when(jnp.logical_not(is_a))
        def _():
            pltpu.sync_copy(rb_hbm.at[pl.ds(base, _HALF_ELEMS)], r_v)

    def zero_hist(t, _):
        hist2d[pl.ds(t * _LANES, _LANES)] = jnp.zeros((_LANES,), jnp.int32)
        return 0

    lax.fori_loop(0, _BPAD, zero_hist, 0)

    def p1_step(j, _):
        gtv = gt_v[pl.ds(j * _LANES, _LANES)]
        rv = r_v[pl.ds(j * _LANES, _LANES)]
        zm = gtv == 0
        m = jnp.where(is_a, zm, jnp.logical_not(zm))
        bucket = lax.shift_right_logical(rv, _BSHIFT)
        plsc.addupdate_scatter(hist2d, [bucket * _LANES + lanes], ones, mask=m)
        return 0

    for h in range(2):
        load_half(h)
        lax.fori_loop(0, _STEPS, p1_step, 0)

    def reduce_rows(g, _):
        def one(t, acc):
            s = jnp.sum(hist2d[pl.ds((g * _LANES + t) * _LANES, _LANES)])
            return jnp.where(lanes == t, s, acc)

        hist1d[pl.ds(g * _LANES, _LANES)] = lax.fori_loop(
            0, _LANES, one, jnp.zeros((_LANES,), jnp.int32))
        return 0

    lax.fori_loop(0, _BPAD // _LANES, reduce_rows, 0)
    pltpu.sync_copy(hist1d, sh_hist.at[sid])
    plsc.subcore_barrier()

    @pl.when(sid == 0)
    def _():
        pltpu.sync_copy(sh_hist, all_hist)

        def merge_find(t, carry):
            found_t, kp, run = carry
            acc = jnp.zeros((_LANES,), jnp.int32)

            def addt(s, a):
                return a + all_hist[s, pl.ds(t * _LANES, _LANES)]

            acc = lax.fori_loop(0, _NS, addt, acc)
            s16 = plsc.cumsum(acc)
            tot = _lane_extract(s16, _LANES - 1)
            hist1d[pl.ds(t * _LANES, _LANES)] = acc
            return found_t, kp, run + tot

        _, _, m_tot = lax.fori_loop(0, _BPAD // _LANES, merge_find,
                                    (jnp.int32(-1), jnp.int32(0),
                                     jnp.int32(0)))
        num_mask = m_tot
        num_samples = jnp.minimum(
            jnp.minimum(num_mask, _N - num_mask), _EXPECTED)
        k = jnp.where(is_a, num_samples // 2, num_samples)

        def find(t, carry):
            found_t, kp, run = carry
            v = hist1d[pl.ds(t * _LANES, _LANES)]
            s16 = plsc.cumsum(v)
            tot = _lane_extract(s16, _LANES - 1)
            cond = (run + s16) >= k
            idx = jnp.min(jnp.where(cond, lanes, _LANES))
            hit = jnp.logical_and(found_t < 0, idx < _LANES)
            c_before = run + _lane_extract(s16, idx) - _lane_extract(v, idx)
            found_t = jnp.where(hit, t * _LANES + idx, found_t)
            kp = jnp.where(hit, k - c_before, kp)
            return found_t, kp, run + tot

        bt, kp, _ = lax.fori_loop(0, _BPAD // _LANES, find,
                                  (jnp.int32(-1), jnp.int32(0),
                                   jnp.int32(0)))
        ctl_v[...] = jnp.broadcast_to(bt, (_LANES,))
        pltpu.sync_copy(ctl_v, sh_ctl)
        hist1d[pl.ds(0, _LANES)] = (
            k * jnp.where(lanes == 0, 1, 0) + kp * jnp.where(lanes == 1, 1, 0))

    plsc.subcore_barrier()
    pltpu.sync_copy(sh_ctl, ctl_v)
    bt_vec = ctl_v[...]

    def zero_bm(t, _):
        bitmap[pl.ds(t * _LANES, _LANES)] = jnp.zeros((_LANES,), jnp.int32)
        return 0

    lax.fori_loop(0, _SUB // _LANES, zero_bm, 0)

    def p2_step(j, _):
        gtv = gt_v[pl.ds(j * _LANES, _LANES)]
        rv = r_v[pl.ds(j * _LANES, _LANES)]
        zm = gtv == 0
        m = jnp.where(is_a, zm, jnp.logical_not(zm))
        bucket = lax.shift_right_logical(rv, _BSHIFT)
        m2 = jnp.logical_and(m, bucket == bt_vec)
        subr = jnp.bitwise_and(rv, _SUB - 1)
        plsc.store_scatter(bitmap, [subr], ones, mask=m2)
        return 0

    for h in range(2):
        load_half(h)
        lax.fori_loop(0, _STEPS, p2_step, 0)

    pltpu.sync_copy(bitmap, sh_bm.at[sid])
    plsc.subcore_barrier()

    @pl.when(sid == 0)
    def _():
        pltpu.sync_copy(sh_bm, bm_all)
        kctl = hist1d[pl.ds(0, _LANES)]
        k = _lane_extract(kctl, 0)
        kp = _lane_extract(kctl, 1)
        bt = _lane_extract(bt_vec, 0)

        def scan(g, carry):
            pos, run = carry
            acc = jnp.zeros((_LANES,), jnp.int32)

            def adds(s, a):
                return a + bm_all[s, pl.ds(g * _LANES, _LANES)]

            acc = lax.fori_loop(0, _NS, adds, acc)
            s16 = plsc.cumsum(acc)
            tot = _lane_extract(s16, _LANES - 1)
            cond = (run + s16) >= kp
            idx = jnp.min(jnp.where(cond, lanes, _LANES))
            hit = jnp.logical_and(pos < 0, idx < _LANES)
            pos = jnp.where(hit, g * _LANES + idx, pos)
            return pos, run + tot

        pos, _ = lax.fori_loop(0, _SUB // _LANES, scan,
                               (jnp.int32(-1), jnp.int32(0)))
        rho = jnp.where(k > 0, bt * _SUB + pos + 1, 0)
        ctl_v[...] = jnp.broadcast_to(rho, (_LANES,))
        pltpu.sync_copy(ctl_v, out_hbm.at[cid])


def _make_sel_kernel():
    mesh = plsc.VectorSubcoreMesh(core_axis_name="c", subcore_axis_name="s")
    return pl.kernel(
        _sel_body,
        out_type=jax.ShapeDtypeStruct((2, _LANES), jnp.int32),
        mesh=mesh,
        scratch_types=[
            pltpu.VMEM((_HALF_ELEMS,), jnp.int32),
            pltpu.VMEM((_HALF_ELEMS,), jnp.int32),
            pltpu.VMEM((_BPAD * _LANES,), jnp.int32),
            pltpu.VMEM((_BPAD,), jnp.int32),
            pltpu.VMEM((_NS, _BPAD), jnp.int32),
            pltpu.VMEM((_SUB,), jnp.int32),
            pltpu.VMEM((_NS, _SUB), jnp.int32),
            pltpu.VMEM((_LANES,), jnp.int32),
            pltpu.VMEM_SHARED((_NS, _BPAD), jnp.int32),
            pltpu.VMEM_SHARED((_NS, _SUB), jnp.int32),
            pltpu.VMEM_SHARED((_LANES,), jnp.int32),
        ],
        compiler_params=pltpu.CompilerParams(needs_layout_passes=False),
    )


def _nll_body(pred_ref, gt_ref, out_ref):
    x = pred_ref[0]
    m = jnp.max(x, axis=0)
    s = jnp.sum(jnp.exp(x - m[None, :, :]), axis=0)
    lse = m + jnp.log(s)
    labels = gt_ref[0]
    cls = jax.lax.broadcasted_iota(jnp.int32, x.shape, 0)
    xl = jnp.sum(jnp.where(cls == labels[None, :, :], x, 0.0), axis=0)
    out_ref[0] = lse - xl


def _loss_body(nll_ref, gt_ref, ra_ref, rb_ref, sel_ref, out_ref):
    gt = gt_ref[...]
    z = gt == 0
    num_zero = jnp.sum(jnp.where(z, 1, 0))
    num_non_zero = _N - num_zero
    num_samples = jnp.minimum(
        jnp.minimum(num_zero, num_non_zero), _EXPECTED)
    half = num_samples // 2

    rho_a = sel_ref[0, 0]
    rho_b = sel_ref[1, 0]

    nll = nll_ref[...]
    s1 = jnp.sum(jnp.where(z & (ra_ref[...] < rho_a), nll, 0.0))
    s2 = jnp.sum(jnp.where((~z) & (rb_ref[...] < rho_b), nll, 0.0))
    loss1 = s1 / half.astype(jnp.float32)
    loss2 = s2 / num_samples.astype(jnp.float32)
    sampled = _LAMBDS[0] * loss1 + _LAMBDS[1] * loss2
    full = jnp.sum(nll) / jnp.float32(_N)
    result = jnp.where(num_samples > 0, sampled, full)
    out_ref[...] = jnp.broadcast_to(result, (1, 1))


def kernel(pred, gt):
    rank_a, rank_b = _rank_constants()
    ra = jnp.asarray(rank_a)
    rb = jnp.asarray(rank_b)
    gt = gt.astype(jnp.int32)
    gt_flat = gt.reshape(_N)

    sel = _make_sel_kernel()(gt_flat, ra, rb)

    nll = pl.pallas_call(
        _nll_body,
        grid=(_B, _H // _ROW_BLOCK),
        in_specs=[
            pl.BlockSpec((1, _C, _ROW_BLOCK, _W), lambda b, y: (b, 0, y, 0)),
            pl.BlockSpec((1, _ROW_BLOCK, _W), lambda b, y: (b, y, 0)),
        ],
        out_specs=pl.BlockSpec((1, _ROW_BLOCK, _W), lambda b, y: (b, y, 0)),
        out_shape=jax.ShapeDtypeStruct((_B, _H, _W), jnp.float32),
    )(pred, gt)

    loss = pl.pallas_call(
        _loss_body,
        in_specs=[
            pl.BlockSpec((_B, _H, _W), lambda: (0, 0, 0)),
            pl.BlockSpec((_B, _H, _W), lambda: (0, 0, 0)),
            pl.BlockSpec((_B, _H, _W), lambda: (0, 0, 0)),
            pl.BlockSpec((_B, _H, _W), lambda: (0, 0, 0)),
            pl.BlockSpec(memory_space=pltpu.SMEM),
        ],
        out_specs=pl.BlockSpec((1, 1), lambda: (0, 0)),
        out_shape=jax.ShapeDtypeStruct((1, 1), jnp.float32),
    )(nll, gt, ra.reshape(_B, _H, _W), rb.reshape(_B, _H, _W), sel)

    return loss[0, 0]


# speedup vs baseline: 1.0092x; 1.0092x over previous
import functools

import jax
import jax.numpy as jnp
import numpy as np
from jax import lax
from jax.experimental import pallas as pl
from jax.experimental.pallas import tpu as pltpu
from jax.experimental.pallas import tpu_sc as plsc

_B, _C, _H, _W = 4, 96, 384, 384
_N = _B * _H * _W
_SAMPLES_PER_IM = 5000
_EXPECTED = _SAMPLES_PER_IM * _B
_LAMBDS = (1.0 / 6.0, 5.0 / 6.0)
_ROW_BLOCK = 48

_NS, _LANES = 16, 16
_TILE_ELEMS = _N // _NS
_HALF_ELEMS = _TILE_ELEMS // 2
_STEPS = _HALF_ELEMS // _LANES
_BSHIFT = 12
_NBUCK = _N >> _BSHIFT
_BPAD = 160
_SUB = 1 << _BSHIFT


@functools.lru_cache(maxsize=1)
def _rank_constants():
    with jax.ensure_compile_time_eval():
        skey = jax.random.key(42)
        ka, kb = jax.random.split(skey)
        ranks = []
        for k in (ka, kb):
            g = jax.random.gumbel(k, (_N,), dtype=jnp.float32)
            perm = jnp.argsort(-g, stable=True)
            rank = jnp.zeros((_N,), jnp.int32).at[perm].set(
                jnp.arange(_N, dtype=jnp.int32))
            ranks.append(np.asarray(rank))
    return tuple(ranks)


def _lane_extract(vec, lane_idx):
    lanes = lax.iota(jnp.int32, _LANES)
    return jnp.sum(jnp.where(lanes == lane_idx, vec, 0))


def _sel_body(gt_hbm, ra_hbm, rb_hbm, out_hbm,
              gt_v, r_v, hist2d, hist1d, all_hist, bitmap, bm_all, ctl_v,
              sh_hist, sh_bm, sh_ctl):
    cid = lax.axis_index("c")
    sid = lax.axis_index("s")
    is_a = cid == 0
    lanes = lax.iota(jnp.int32, _LANES)
    ones = jnp.full((_LANES,), 1, jnp.int32)

    def load_half(h):
        base = sid * _TILE_ELEMS + h * _HALF_ELEMS
        pltpu.sync_copy(gt_hbm.at[pl.ds(base, _HALF_ELEMS)], gt_v)

        @pl.when(is_a)
        def _():
            pltpu.sync_copy(ra_hbm.at[pl.ds(base, _HALF_ELEMS)], r_v)

        @pl.when(jnp.logical_not(is_a))
        def _():
            pltpu.sync_copy(rb_hbm.at[pl.ds(base, _HALF_ELEMS)], r_v)

    def zero_hist(t, _):
        hist2d[pl.ds(t * _LANES, _LANES)] = jnp.zeros((_LANES,), jnp.int32)
        return 0

    lax.fori_loop(0, _BPAD, zero_hist, 0)

    def p1_step(j, _):
        for u in range(4):
            gtv = gt_v[pl.ds((j * 4 + u) * _LANES, _LANES)]
            rv = r_v[pl.ds((j * 4 + u) * _LANES, _LANES)]
            zm = gtv == 0
            m = jnp.where(is_a, zm, jnp.logical_not(zm))
            bucket = lax.shift_right_logical(rv, _BSHIFT)
            plsc.addupdate_scatter(
                hist2d, [bucket * _LANES + lanes], ones, mask=m)
        return 0

    for h in range(2):
        load_half(h)
        lax.fori_loop(0, _STEPS // 4, p1_step, 0)

    def reduce_rows(g, _):
        def one(t, acc):
            s = jnp.sum(hist2d[pl.ds((g * _LANES + t) * _LANES, _LANES)])
            return jnp.where(lanes == t, s, acc)

        hist1d[pl.ds(g * _LANES, _LANES)] = lax.fori_loop(
            0, _LANES, one, jnp.zeros((_LANES,), jnp.int32))
        return 0

    lax.fori_loop(0, _BPAD // _LANES, reduce_rows, 0)
    pltpu.sync_copy(hist1d, sh_hist.at[sid])
    plsc.subcore_barrier()

    @pl.when(sid == 0)
    def _():
        pltpu.sync_copy(sh_hist, all_hist)

        def merge_find(t, carry):
            found_t, kp, run = carry
            acc = all_hist[0, pl.ds(t * _LANES, _LANES)]
            for u in range(1, _NS):
                acc = acc + all_hist[u, pl.ds(t * _LANES, _LANES)]
            s16 = plsc.cumsum(acc)
            tot = _lane_extract(s16, _LANES - 1)
            hist1d[pl.ds(t * _LANES, _LANES)] = acc
            return found_t, kp, run + tot

        _, _, m_tot = lax.fori_loop(0, _BPAD // _LANES, merge_find,
                                    (jnp.int32(-1), jnp.int32(0),
                                     jnp.int32(0)))
        num_mask = m_tot
        num_samples = jnp.minimum(
            jnp.minimum(num_mask, _N - num_mask), _EXPECTED)
        k = jnp.where(is_a, num_samples // 2, num_samples)

        def find(t, carry):
            found_t, kp, run = carry
            v = hist1d[pl.ds(t * _LANES, _LANES)]
            s16 = plsc.cumsum(v)
            tot = _lane_extract(s16, _LANES - 1)
            cond = (run + s16) >= k
            idx = jnp.min(jnp.where(cond, lanes, _LANES))
            hit = jnp.logical_and(found_t < 0, idx < _LANES)
            c_before = run + _lane_extract(s16, idx) - _lane_extract(v, idx)
            found_t = jnp.where(hit, t * _LANES + idx, found_t)
            kp = jnp.where(hit, k - c_before, kp)
            return found_t, kp, run + tot

        bt, kp, _ = lax.fori_loop(0, _BPAD // _LANES, find,
                                  (jnp.int32(-1), jnp.int32(0),
                                   jnp.int32(0)))
        ctl_v[...] = jnp.broadcast_to(bt, (_LANES,))
        pltpu.sync_copy(ctl_v, sh_ctl)
        hist1d[pl.ds(0, _LANES)] = (
            k * jnp.where(lanes == 0, 1, 0) + kp * jnp.where(lanes == 1, 1, 0))

    plsc.subcore_barrier()
    pltpu.sync_copy(sh_ctl, ctl_v)
    bt_vec = ctl_v[...]

    def zero_bm(t, _):
        bitmap[pl.ds(t * _LANES, _LANES)] = jnp.zeros((_LANES,), jnp.int32)
        return 0

    lax.fori_loop(0, _SUB // _LANES, zero_bm, 0)

    def p2_step(j, _):
        for u in range(4):
            gtv = gt_v[pl.ds((j * 4 + u) * _LANES, _LANES)]
            rv = r_v[pl.ds((j * 4 + u) * _LANES, _LANES)]
            zm = gtv == 0
            m = jnp.where(is_a, zm, jnp.logical_not(zm))
            bucket = lax.shift_right_logical(rv, _BSHIFT)
            m2 = jnp.logical_and(m, bucket == bt_vec)
            subr = jnp.bitwise_and(rv, _SUB - 1)
            plsc.store_scatter(bitmap, [subr], ones, mask=m2)
        return 0

    for h in range(2):
        load_half(h)
        lax.fori_loop(0, _STEPS // 4, p2_step, 0)

    pltpu.sync_copy(bitmap, sh_bm.at[sid])
    plsc.subcore_barrier()

    @pl.when(sid == 0)
    def _():
        pltpu.sync_copy(sh_bm, bm_all)
        kctl = hist1d[pl.ds(0, _LANES)]
        k = _lane_extract(kctl, 0)
        kp = _lane_extract(kctl, 1)
        bt = _lane_extract(bt_vec, 0)

        def scan(g, carry):
            pos, run = carry
            acc = bm_all[0, pl.ds(g * _LANES, _LANES)]
            for u in range(1, _NS):
                acc = acc + bm_all[u, pl.ds(g * _LANES, _LANES)]
            s16 = plsc.cumsum(acc)
            tot = _lane_extract(s16, _LANES - 1)
            cond = (run + s16) >= kp
            idx = jnp.min(jnp.where(cond, lanes, _LANES))
            hit = jnp.logical_and(pos < 0, idx < _LANES)
            pos = jnp.where(hit, g * _LANES + idx, pos)
            return pos, run + tot

        pos, _ = lax.fori_loop(0, _SUB // _LANES, scan,
                               (jnp.int32(-1), jnp.int32(0)))
        rho = jnp.where(k > 0, bt * _SUB + pos + 1, 0)
        ctl_v[...] = jnp.broadcast_to(rho, (_LANES,))
        pltpu.sync_copy(ctl_v, out_hbm.at[cid])


def _make_sel_kernel():
    mesh = plsc.VectorSubcoreMesh(core_axis_name="c", subcore_axis_name="s")
    return pl.kernel(
        _sel_body,
        out_type=jax.ShapeDtypeStruct((2, _LANES), jnp.int32),
        mesh=mesh,
        scratch_types=[
            pltpu.VMEM((_HALF_ELEMS,), jnp.int32),
            pltpu.VMEM((_HALF_ELEMS,), jnp.int32),
            pltpu.VMEM((_BPAD * _LANES,), jnp.int32),
            pltpu.VMEM((_BPAD,), jnp.int32),
            pltpu.VMEM((_NS, _BPAD), jnp.int32),
            pltpu.VMEM((_SUB,), jnp.int32),
            pltpu.VMEM((_NS, _SUB), jnp.int32),
            pltpu.VMEM((_LANES,), jnp.int32),
            pltpu.VMEM_SHARED((_NS, _BPAD), jnp.int32),
            pltpu.VMEM_SHARED((_NS, _SUB), jnp.int32),
            pltpu.VMEM_SHARED((_LANES,), jnp.int32),
        ],
        compiler_params=pltpu.CompilerParams(needs_layout_passes=False),
    )


def _nll_body(pred_ref, gt_ref, out_ref):
    x = pred_ref[0]
    m = jnp.max(x, axis=0)
    s = jnp.sum(jnp.exp(x - m[None, :, :]), axis=0)
    lse = m + jnp.log(s)
    labels = gt_ref[0]
    cls = jax.lax.broadcasted_iota(jnp.int32, x.shape, 0)
    xl = jnp.sum(jnp.where(cls == labels[None, :, :], x, 0.0), axis=0)
    out_ref[0] = lse - xl


def _loss_body(nll_ref, gt_ref, ra_ref, rb_ref, sel_ref, out_ref):
    gt = gt_ref[...]
    z = gt == 0
    num_zero = jnp.sum(jnp.where(z, 1, 0))
    num_non_zero = _N - num_zero
    num_samples = jnp.minimum(
        jnp.minimum(num_zero, num_non_zero), _EXPECTED)
    half = num_samples // 2

    rho_a = sel_ref[0, 0]
    rho_b = sel_ref[1, 0]

    nll = nll_ref[...]
    s1 = jnp.sum(jnp.where(z & (ra_ref[...] < rho_a), nll, 0.0))
    s2 = jnp.sum(jnp.where((~z) & (rb_ref[...] < rho_b), nll, 0.0))
    loss1 = s1 / half.astype(jnp.float32)
    loss2 = s2 / num_samples.astype(jnp.float32)
    sampled = _LAMBDS[0] * loss1 + _LAMBDS[1] * loss2
    full = jnp.sum(nll) / jnp.float32(_N)
    result = jnp.where(num_samples > 0, sampled, full)
    out_ref[...] = jnp.broadcast_to(result, (1, 1))


def kernel(pred, gt):
    rank_a, rank_b = _rank_constants()
    ra = jnp.asarray(rank_a)
    rb = jnp.asarray(rank_b)
    gt = gt.astype(jnp.int32)
    gt_flat = gt.reshape(_N)

    sel = _make_sel_kernel()(gt_flat, ra, rb)

    nll = pl.pallas_call(
        _nll_body,
        grid=(_B, _H // _ROW_BLOCK),
        in_specs=[
            pl.BlockSpec((1, _C, _ROW_BLOCK, _W), lambda b, y: (b, 0, y, 0)),
            pl.BlockSpec((1, _ROW_BLOCK, _W), lambda b, y: (b, y, 0)),
        ],
        out_specs=pl.BlockSpec((1, _ROW_BLOCK, _W), lambda b, y: (b, y, 0)),
        out_shape=jax.ShapeDtypeStruct((_B, _H, _W), jnp.float32),
    )(pred, gt)

    loss = pl.pallas_call(
        _loss_body,
        in_specs=[
            pl.BlockSpec((_B, _H, _W), lambda: (0, 0, 0)),
            pl.BlockSpec((_B, _H, _W), lambda: (0, 0, 0)),
            pl.BlockSpec((_B, _H, _W), lambda: (0, 0, 0)),
            pl.BlockSpec((_B, _H, _W), lambda: (0, 0, 0)),
            pl.BlockSpec(memory_space=pltpu.SMEM),
        ],
        out_specs=pl.BlockSpec((1, 1), lambda: (0, 0)),
        out_shape=jax.ShapeDtypeStruct((1, 1), jnp.float32),
    )(nll, gt, ra.reshape(_B, _H, _W), rb.reshape(_B, _H, _W), sel)

    return loss[0, 0]
